# Initial kernel scaffold; baseline (speedup 1.0000x reference)
import jax
import jax.numpy as jnp
from jax.experimental import pallas as pl


def kernel(x, edge_index, W0, b0, Wfc, bfc, W1, b1, W2, b2, W3, b3, Wq1, bq1, p, Wq2, bq2, Wlin, blin):
    raise NotImplementedError("write your pallas kernel here")


# speedup vs baseline: 3.2388x; 3.2388x over previous
import functools
import jax
import jax.numpy as jnp
from jax import lax
from jax.experimental import pallas as pl
from jax.experimental.pallas import tpu as pltpu
from jax.experimental.pallas import tpu_sc as plsc

_N = 10000
_G = 8
_S = _N // _G
_E = 160000
_HID = 256
_KSEL = 10
_KP = _S // 2

_NSC = 2
_NTEC = 16
_NWK = _NSC * _NTEC
_RIDX = 40
_EPAD = _NWK * _RIDX * 128
_NPAD = 10240
_TROWS = _NPAD // _NTEC
_BR = 1280


def _sc_agg(tables, srcw, dstw, zrow):
    C = len(tables)
    D = tables[0].shape[1]
    mesh = plsc.VectorSubcoreMesh(
        core_axis_name="c", subcore_axis_name="s",
        num_cores=_NSC, num_subcores=_NTEC)

    def body(*refs):
        tabs = refs[:C]
        srch, dsth, zr, out = refs[C:C + 4]
        src_v, dst_v, rows_v, shared = refs[C + 4:]
        cid = lax.axis_index("c")
        sid = lax.axis_index("s")
        w = sid * _NSC + cid
        pltpu.sync_copy(srch.at[w], src_v)
        pltpu.sync_copy(dsth.at[w], dst_v)
        row0 = sid * _TROWS
        for c in range(C):
            pltpu.sync_copy(zr, shared.at[pl.ds(row0, _TROWS)])
            plsc.subcore_barrier()

            def step(j, carry, c=c):
                pltpu.sync_copy(tabs[c].at[src_v.at[j]], rows_v)
                pltpu.sync_copy(rows_v, shared.at[dst_v.at[j]], add=True)
                return carry

            lax.fori_loop(0, _RIDX, step, 0)
            plsc.subcore_barrier()
            pltpu.sync_copy(shared.at[pl.ds(row0, _TROWS)],
                            out.at[cid, c, pl.ds(row0, _TROWS)])
            plsc.subcore_barrier()

    f = pl.kernel(
        body,
        out_type=jax.ShapeDtypeStruct((_NSC, C, _NPAD, D), jnp.float32),
        mesh=mesh,
        scratch_types=[
            pltpu.VMEM((_RIDX, 128), jnp.int32),
            pltpu.VMEM((_RIDX, 128), jnp.int32),
            pltpu.VMEM((128, D), jnp.float32),
            pltpu.VMEM_SHARED((_NPAD, D), jnp.float32),
        ],
    )
    return f(*tables, srcw, dstw, zrow)


def _rows_spec(shape, br):
    bs = list(shape)
    ax = len(shape) - 2
    bs[ax] = br
    n = len(shape)

    def imap(i, ax=ax, n=n):
        idx = [0] * n
        idx[ax] = i
        return tuple(idx)

    return pl.BlockSpec(tuple(bs), imap)


def _full_spec(shape):
    n = len(shape)
    return pl.BlockSpec(tuple(shape), lambda i: (0,) * n)


def _rows_call(fn, row_ins, full_ins, out_shapes, br, rows):
    grid = rows // br
    in_specs = ([_rows_spec(a.shape, br) for a in row_ins]
                + [_full_spec(a.shape) for a in full_ins])
    out_shape = [jax.ShapeDtypeStruct(s, jnp.float32) for s in out_shapes]
    out_specs = [_rows_spec(s, br) for s in out_shapes]
    return pl.pallas_call(
        fn,
        grid=(grid,),
        in_specs=in_specs,
        out_specs=out_specs,
        out_shape=out_shape,
    )(*row_ins, *full_ins)


def _mm(x, w):
    return jnp.dot(x, w, preferred_element_type=jnp.float32)


def kernel(x, edge_index, W0, b0, Wfc, bfc, W1, b1, W2, b2, W3, b3,
           Wq1, bq1, p, Wq2, bq2, Wlin, blin):
    f32 = jnp.float32

    src = edge_index[0]
    dst = edge_index[1]
    padn = jnp.full((_EPAD - _E,), _N, jnp.int32)
    srcw = jnp.concatenate([src, padn]).reshape(_NWK, _RIDX, 128)
    dstw = jnp.concatenate([dst, padn]).reshape(_NWK, _RIDX, 128)
    z128 = jnp.zeros((_TROWS, 128), f32)
    ones128 = jnp.ones((_NPAD, 128), f32)

    xpd = jnp.zeros((_NPAD, 128), f32).at[:_N].set(x)
    u = jax.random.uniform(jax.random.key(42), (_N, _KSEL * 3), f32,
                           1e-6, 1.0 - 1e-6)
    gum = -jnp.log(-jnp.log(u))

    W0c = [W0[:, :128], W0[:, 128:]]
    W1r = [W1[:128], W1[128:]]
    W1c = [W1[:, :128], W1[:, 128:]]
    W2a = W2[:_HID]
    W2b = W2[_HID:]
    W2ar = [W2a[:128], W2a[128:]]
    Wq1p = jnp.zeros((128, 256), f32).at[:, :250].set(Wq1)
    Wq1c = [Wq1p[:, :128], Wq1p[:, 128:]]
    bq1p = jnp.zeros((1, 256), f32).at[:, :250].set(bq1[None, :])
    ppad = jnp.zeros((256, 1), f32).at[:250, 0].set(p)
    Wq2p = jnp.zeros((256, 32), f32).at[:250].set(Wq2)
    Wq2r = [Wq2p[:128], Wq2p[128:]]
    b0r = b0[None, :]
    b1r = b1[None, :]
    b2r = b2[None, :]
    bq2r = bq2[None, :]
    blinr = blin[None, :]

    degp = _sc_agg([ones128], srcw, dstw, z128)

    def t0(degp_r, x_r, w0a, w0b, dinv_o, sl_o, h0_o, h1_o, hs0_o, hs1_o):
        deg = 1.0 + degp_r[0, 0, :, 0:1] + degp_r[1, 0, :, 0:1]
        dinv = lax.rsqrt(deg)
        dinv_o[...] = dinv
        sl_o[...] = 1.0 / deg
        xa = x_r[...]
        ha = _mm(xa, w0a[...])
        hb = _mm(xa, w0b[...])
        h0_o[...] = ha
        h1_o[...] = hb
        hs0_o[...] = ha * dinv
        hs1_o[...] = hb * dinv

    dinv, sl, h_a, h_b, hs_a, hs_b = _rows_call(
        t0, [degp, xpd], [W0c[0], W0c[1]],
        [(_NPAD, 1), (_NPAD, 1), (_NPAD, 128), (_NPAD, 128),
         (_NPAD, 128), (_NPAD, 128)], _BR, _NPAD)

    agg0 = _sc_agg([hs_a, hs_b], srcw, dstw, z128)

    def _finish_mm2(aggp, ha, hb, dinv_a, sl_a, wra, wrb, bias,
                    n0_o, n1_o, y0_o, y1_o, ys0_o, ys1_o):
        dv = dinv_a[...]
        s = sl_a[...]
        b_ = bias[...]
        c0 = jax.nn.relu(dv * (aggp[0, 0] + aggp[1, 0]) + ha[...] * s
                         + b_[:, :128])
        c1 = jax.nn.relu(dv * (aggp[0, 1] + aggp[1, 1]) + hb[...] * s
                         + b_[:, 128:])
        n0_o[...] = c0
        n1_o[...] = c1
        y0 = _mm(c0, wra[...][:, :128]) + _mm(c1, wrb[...][:, :128])
        y1 = _mm(c0, wra[...][:, 128:]) + _mm(c1, wrb[...][:, 128:])
        y0_o[...] = y0
        y1_o[...] = y1
        ys0_o[...] = y0 * dv
        ys1_o[...] = y1 * dv

    h0_a, h0_b, y1_a, y1_b, y1s_a, y1s_b = _rows_call(
        _finish_mm2,
        [agg0, h_a, h_b, dinv, sl], [W1r[0], W1r[1], b0r],
        [(_NPAD, 128)] * 6, _BR, _NPAD)

    def _graph_call(fn, row_ins, full_ins, out_shapes):
        rs = []
        in_specs = []
        for a in row_ins:
            if a.ndim == 2:
                d = a.shape[1]
                rs.append(a.reshape(_G, _S, d))
                in_specs.append(pl.BlockSpec((1, _S, d),
                                             lambda i: (i, 0, 0)))
            else:
                two, C, n, D = a.shape
                rs.append(a.reshape(two, C, _G, _S, D))
                in_specs.append(pl.BlockSpec((two, C, 1, _S, D),
                                             lambda i: (0, 0, i, 0, 0)))
        in_specs += [_full_spec(a.shape) for a in full_ins]
        out_shape = []
        out_specs = []
        final = []
        for s in out_shapes:
            if s[0] == _G:
                out_shape.append(jax.ShapeDtypeStruct((_G, 1, s[1]),
                                                      jnp.float32))
                out_specs.append(pl.BlockSpec((1, 1, s[1]),
                                              lambda i: (i, 0, 0)))
                final.append((_G, s[1]))
            else:
                out_shape.append(jax.ShapeDtypeStruct((_G, _S, s[1]),
                                                      jnp.float32))
                out_specs.append(pl.BlockSpec((1, _S, s[1]),
                                              lambda i: (i, 0, 0)))
                final.append((s[0], s[1]))
        outs = pl.pallas_call(
            fn, grid=(_G,), in_specs=in_specs, out_specs=out_specs,
            out_shape=out_shape)(*rs, *full_ins)
        return tuple(o.reshape(f) for o, f in zip(outs, final))

    def tg(h0a, h0b, wfc, bfc_, w2b, cv_o):
        g0 = jnp.max(h0a[0], axis=0, keepdims=True)
        g1 = jnp.max(h0b[0], axis=0, keepdims=True)
        gi = (_mm(g0, wfc[...][:128]) + _mm(g1, wfc[...][128:]) + bfc_[...])
        cv_o[0] = _mm(gi, w2b[...])

    (cvec,) = _graph_call(tg, [h0_a[:_N], h0_b[:_N]],
                          [Wfc, bfc[None, :], W2b], [(_G, _HID)])
    cfull = jnp.zeros((_NPAD, _HID), f32).at[:_N].set(
        jnp.repeat(cvec, _S, axis=0))

    agg1 = _sc_agg([y1s_a, y1s_b], srcw, dstw, z128)
    h1_a, h1_b, y2_a, y2_b, y2s_a, y2s_b = _rows_call(
        _finish_mm2, [agg1, y1_a, y1_b, dinv, sl],
        [W1r[0], W1r[1], b1r], [(_NPAD, 128)] * 6, _BR, _NPAD)

    agg2 = _sc_agg([y2s_a, y2s_b], srcw, dstw, z128)

    def t3(aggp, ha, hb, dinv_a, sl_a, cf, wra, wrb, bias,
           y0_o, y1_o, ys0_o, ys1_o):
        dv = dinv_a[...]
        s = sl_a[...]
        b_ = bias[...]
        c0 = jax.nn.relu(dv * (aggp[0, 0] + aggp[1, 0]) + ha[...] * s
                         + b_[:, :128])
        c1 = jax.nn.relu(dv * (aggp[0, 1] + aggp[1, 1]) + hb[...] * s
                         + b_[:, 128:])
        cfl = cf[...]
        y0 = _mm(c0, wra[...][:, :128]) + _mm(c1, wrb[...][:, :128]) \
            + cfl[:, :128]
        y1 = _mm(c0, wra[...][:, 128:]) + _mm(c1, wrb[...][:, 128:]) \
            + cfl[:, 128:]
        y0_o[...] = y0
        y1_o[...] = y1
        ys0_o[...] = y0 * dv
        ys1_o[...] = y1 * dv

    h2p_a, h2p_b, h2ps_a, h2ps_b = _rows_call(
        t3, [agg2, y2_a, y2_b, dinv, sl, cfull],
        [W2ar[0], W2ar[1], b1r], [(_NPAD, 128)] * 4, _BR, _NPAD)

    agg3 = _sc_agg([h2ps_a, h2ps_b], srcw, dstw, z128)

    def t4b(aggp, ha, hb, dinv_a, sl_a, w3, bias, lg8_o, lg_o):
        dv = dinv_a[...]
        s = sl_a[...]
        b_ = bias[...]
        c0 = jax.nn.relu(dv * (aggp[0, 0] + aggp[1, 0]) + ha[...] * s
                         + b_[:, :128])
        c1 = jax.nn.relu(dv * (aggp[0, 1] + aggp[1, 1]) + hb[...] * s
                         + b_[:, 128:])
        lg = _mm(c0, w3[...][:128]) + _mm(c1, w3[...][128:])
        lg8_o[...] = jnp.broadcast_to(lg * dv, (lg.shape[0], 128))
        lg_o[...] = lg

    lg8, lg = _rows_call(t4b, [agg3, h2p_a, h2p_b, dinv, sl],
                         [W3, b2r], [(_NPAD, 128), (_NPAD, 1)], _BR, _NPAD)

    agg4 = _sc_agg([lg8], srcw, dstw, z128)

    def t5(aggp, lg_r, dinv_a, sl_a, gum_r, x_r, wq1a, wq1b, b3_,
           q1h0_o, q1h1_o, q1hs0_o, q1hs1_o):
        dv = dinv_a[0]
        ag = aggp[0, 0, 0] + aggp[1, 0, 0]
        logit = dv * ag[:, 0:1] + lg_r[0] * sl_a[0] + b3_[...]
        noisy = gum_r[0] + logit
        m = jnp.max(noisy, axis=0, keepdims=True)
        e = jnp.exp(noisy - m)
        se = jnp.sum(e, axis=0, keepdims=True)
        T = jnp.max(e / se, axis=1, keepdims=True)
        xm = x_r[0] * T
        qa = _mm(xm, wq1a[...])
        qb = _mm(xm, wq1b[...])
        q1h0_o[0] = qa
        q1h1_o[0] = qb
        q1hs0_o[0] = qa * dv
        q1hs1_o[0] = qb * dv

    q1h_a, q1h_b, q1hs_a, q1hs_b = _graph_call(
        t5, [agg4[:, :, :_N], lg[:_N], dinv[:_N], sl[:_N], gum, x],
        [Wq1c[0], Wq1c[1], b3[None, :]],
        [(_N, 128), (_N, 128), (_N, 128), (_N, 128)])
    q1hs_a = jnp.zeros((_NPAD, 128), f32).at[:_N].set(q1hs_a)
    q1hs_b = jnp.zeros((_NPAD, 128), f32).at[:_N].set(q1hs_b)

    agg5 = _sc_agg([q1hs_a, q1hs_b], srcw, dstw, z128)

    def t6(aggp, ha, hb, dinv_a, sl_a, bias, pp, wq2a, wq2b,
           keep8_o, hq2_o, keep_o):
        dv = dinv_a[0]
        s = sl_a[0]
        b_ = bias[...]
        c0 = dv * (aggp[0, 0, 0] + aggp[1, 0, 0]) + ha[0] * s + b_[:, :128]
        c1 = dv * (aggp[0, 1, 0] + aggp[1, 1, 0]) + hb[0] * s + b_[:, 128:]
        ppv = pp[...]
        pnorm = lax.rsqrt(jnp.sum(ppv * ppv))
        score = (_mm(c0, ppv[:128]) + _mm(c1, ppv[128:])) * pnorm

        bits = lax.bitcast_convert_type(score, jnp.uint32)
        key = jnp.where(bits >> 31 == jnp.uint32(1), ~bits,
                        bits | jnp.uint32(0x80000000))

        def bit_step(i, carry):
            prefix, need = carry
            b_pos = (31 - i).astype(jnp.uint32)
            cand = (prefix >> b_pos) | jnp.uint32(1)
            cnt = jnp.sum(jnp.where((key >> b_pos) == cand, 1, 0))
            take = cnt >= need
            prefix = jnp.where(take, prefix | (jnp.uint32(1) << b_pos),
                               prefix)
            need = jnp.where(take, need, need - cnt)
            return prefix, need

        prefix, _need = lax.fori_loop(
            0, 32, bit_step, (jnp.uint32(0), jnp.int32(_KP)))
        cnt_gt = jnp.sum(jnp.where(key > prefix, 1, 0))
        extra = _KP - cnt_gt
        idx = lax.broadcasted_iota(jnp.int32, key.shape, 0)
        tie = key == prefix

        def ib_step(i, carry):
            lo, hi = carry
            mid = (lo + hi) // 2
            c = jnp.sum(jnp.where(tie & (idx <= mid), 1, 0))
            good = c >= extra
            hi = jnp.where(good, mid, hi)
            lo = jnp.where(good, lo, mid + 1)
            return lo, hi

        lo, _hi = lax.fori_loop(0, 11, ib_step,
                                (jnp.int32(0), jnp.int32(_S - 1)))
        keep = jnp.where((key > prefix) | (tie & (idx <= lo)), 1.0, 0.0)
        keep_o[0] = keep[:, 0:1]
        keep8_o[0] = jnp.broadcast_to(keep[:, 0:1], (keep.shape[0], 128))
        w = jnp.tanh(score)
        xp0 = jax.nn.relu(c0 * w) * keep[:, 0:1]
        xp1 = jax.nn.relu(c1 * w) * keep[:, 0:1]
        hq2_o[0] = _mm(xp0, wq2a[...]) + _mm(xp1, wq2b[...])

    keep8, hq2, keep = _graph_call(
        t6, [agg5[:, :, :_N], q1h_a, q1h_b, dinv[:_N], sl[:_N]],
        [bq1p, ppad, Wq2r[0], Wq2r[1]],
        [(_N, 128), (_N, 32), (_N, 1)])
    keep8p = jnp.zeros((_NPAD, 128), f32).at[:_N].set(keep8)

    degq = _sc_agg([keep8p], srcw, dstw, z128)

    def t7(degq_r, hq2_r, keep_r, hq2s_o, dinv0_o, slp_o):
        kp = keep_r[...]
        sj = degq_r[0, 0, :, 0:1] + degq_r[1, 0, :, 0:1]
        dp = 1.0 + kp * sj
        dinv0 = kp * lax.rsqrt(dp)
        dinv0_o[...] = dinv0
        slp_o[...] = 1.0 / dp
        hq2s_o[...] = hq2_r[...] * dinv0

    hq2p = jnp.zeros((_NPAD, 128), f32).at[:_N, :32].set(hq2)
    keepp = jnp.zeros((_NPAD, 1), f32).at[:_N].set(keep)
    hq2sp, dinv0, slp = _rows_call(
        t7, [degq, hq2p, keepp], [],
        [(_NPAD, 128), (_NPAD, 1), (_NPAD, 1)], _BR, _NPAD)

    agg6 = _sc_agg([hq2sp], srcw, dstw, z128)

    def t8(aggp, hq2_r, dinv0_r, slp_r, keep_r, bq2_, wlin, blin_, out_o):
        dv0 = dinv0_r[0]
        q2 = jax.nn.relu(dv0 * (aggp[0, 0, 0] + aggp[1, 0, 0])[:, :32]
                         + hq2_r[0] * slp_r[0] + bq2_[...])
        q2 = q2 * keep_r[0]
        pooled = jnp.sum(q2, axis=0, keepdims=True) * (1.0 / _KP)
        out_o[0] = _mm(pooled, wlin[...]) + blin_[...]

    (out,) = _graph_call(
        t8, [agg6[:, :, :_N], hq2, dinv0[:_N], slp[:_N], keep],
        [bq2r, Wlin, blinr], [(_G, 10)])
    return out


# speedup vs baseline: 3.5330x; 1.0908x over previous
import functools
import jax
import jax.numpy as jnp
from jax import lax
from jax.experimental import pallas as pl
from jax.experimental.pallas import tpu as pltpu
from jax.experimental.pallas import tpu_sc as plsc

_N = 10000
_G = 8
_S = _N // _G
_E = 160000
_HID = 256
_KSEL = 10
_KP = _S // 2

_NSC = 2
_NTEC = 16
_NWK = _NSC * _NTEC
_RIDX = 40
_EPAD = _NWK * _RIDX * 128
_NPAD = 10240
_TROWS = _NPAD // _NTEC
_BR = 1280


def _sc_agg(tables, srcw, dstw, zrow):
    C = len(tables)
    D = tables[0].shape[1]
    mesh = plsc.VectorSubcoreMesh(
        core_axis_name="c", subcore_axis_name="s",
        num_cores=_NSC, num_subcores=_NTEC)

    def body(*refs):
        tabs = refs[:C]
        srch, dsth, zr, out = refs[C:C + 4]
        src_v, dst_v, rows0, rows1, shared, semA, semB = refs[C + 4:]
        cid = lax.axis_index("c")
        sid = lax.axis_index("s")
        w = sid * _NSC + cid
        pltpu.sync_copy(srch.at[w], src_v)
        pltpu.sync_copy(dsth.at[w], dst_v)
        row0 = sid * _TROWS
        for c in range(C):
            pltpu.sync_copy(zr, shared.at[pl.ds(row0, _TROWS)])
            plsc.subcore_barrier()

            pltpu.async_copy(tabs[c].at[src_v.at[0]], rows0, semA)

            def step(i, carry, c=c):
                j0 = 2 * i
                j1 = j0 + 1
                pltpu.async_copy(tabs[c].at[src_v.at[j1]], rows1, semB)
                pltpu.make_async_copy(tabs[c].at[src_v.at[j0]], rows0,
                                      semA).wait()
                pltpu.sync_copy(rows0, shared.at[dst_v.at[j0]], add=True)
                j2 = jnp.minimum(j0 + 2, _RIDX - 2)
                pltpu.async_copy(tabs[c].at[src_v.at[j2]], rows0, semA)
                pltpu.make_async_copy(tabs[c].at[src_v.at[j1]], rows1,
                                      semB).wait()
                pltpu.sync_copy(rows1, shared.at[dst_v.at[j1]], add=True)
                return carry

            lax.fori_loop(0, _RIDX // 2, step, 0)
            pltpu.make_async_copy(tabs[c].at[src_v.at[0]], rows0, semA).wait()
            plsc.subcore_barrier()
            pltpu.sync_copy(shared.at[pl.ds(row0, _TROWS)],
                            out.at[cid, c, pl.ds(row0, _TROWS)])
            plsc.subcore_barrier()

    f = pl.kernel(
        body,
        out_type=jax.ShapeDtypeStruct((_NSC, C, _NPAD, D), jnp.float32),
        mesh=mesh,
        scratch_types=[
            pltpu.VMEM((_RIDX, 128), jnp.int32),
            pltpu.VMEM((_RIDX, 128), jnp.int32),
            pltpu.VMEM((128, D), jnp.float32),
            pltpu.VMEM((128, D), jnp.float32),
            pltpu.VMEM_SHARED((_NPAD, D), jnp.float32),
            pltpu.SemaphoreType.DMA,
            pltpu.SemaphoreType.DMA,
        ],
    )
    return f(*tables, srcw, dstw, zrow)


def _rows_spec(shape, br):
    bs = list(shape)
    ax = len(shape) - 2
    bs[ax] = br
    n = len(shape)

    def imap(i, ax=ax, n=n):
        idx = [0] * n
        idx[ax] = i
        return tuple(idx)

    return pl.BlockSpec(tuple(bs), imap)


def _full_spec(shape):
    n = len(shape)
    return pl.BlockSpec(tuple(shape), lambda i: (0,) * n)


def _rows_call(fn, row_ins, full_ins, out_shapes, br, rows):
    grid = rows // br
    in_specs = ([_rows_spec(a.shape, br) for a in row_ins]
                + [_full_spec(a.shape) for a in full_ins])
    out_shape = [jax.ShapeDtypeStruct(s, jnp.float32) for s in out_shapes]
    out_specs = [_rows_spec(s, br) for s in out_shapes]
    return pl.pallas_call(
        fn,
        grid=(grid,),
        in_specs=in_specs,
        out_specs=out_specs,
        out_shape=out_shape,
    )(*row_ins, *full_ins)


def _mm(x, w):
    return jnp.dot(x, w, preferred_element_type=jnp.float32)


def kernel(x, edge_index, W0, b0, Wfc, bfc, W1, b1, W2, b2, W3, b3,
           Wq1, bq1, p, Wq2, bq2, Wlin, blin):
    f32 = jnp.float32

    src = edge_index[0]
    dst = edge_index[1]
    padn = jnp.full((_EPAD - _E,), _N, jnp.int32)
    srcw = jnp.concatenate([src, padn]).reshape(_NWK, _RIDX, 128)
    dstw = jnp.concatenate([dst, padn]).reshape(_NWK, _RIDX, 128)
    z128 = jnp.zeros((_TROWS, 128), f32)
    ones128 = jnp.ones((_NPAD, 128), f32)

    xpd = jnp.zeros((_NPAD, 128), f32).at[:_N].set(x)
    u = jax.random.uniform(jax.random.key(42), (_N, _KSEL * 3), f32,
                           1e-6, 1.0 - 1e-6)
    gum = -jnp.log(-jnp.log(u))

    W0c = [W0[:, :128], W0[:, 128:]]
    W1r = [W1[:128], W1[128:]]
    W1c = [W1[:, :128], W1[:, 128:]]
    W2a = W2[:_HID]
    W2b = W2[_HID:]
    W2ar = [W2a[:128], W2a[128:]]
    Wq1p = jnp.zeros((128, 256), f32).at[:, :250].set(Wq1)
    Wq1c = [Wq1p[:, :128], Wq1p[:, 128:]]
    bq1p = jnp.zeros((1, 256), f32).at[:, :250].set(bq1[None, :])
    ppad = jnp.zeros((256, 1), f32).at[:250, 0].set(p)
    Wq2p = jnp.zeros((256, 32), f32).at[:250].set(Wq2)
    Wq2r = [Wq2p[:128], Wq2p[128:]]
    b0r = b0[None, :]
    b1r = b1[None, :]
    b2r = b2[None, :]
    bq2r = bq2[None, :]
    blinr = blin[None, :]

    degp = _sc_agg([ones128], srcw, dstw, z128)

    def t0(degp_r, x_r, w0a, w0b, dinv_o, sl_o, h0_o, h1_o, hs0_o, hs1_o):
        deg = 1.0 + degp_r[0, 0, :, 0:1] + degp_r[1, 0, :, 0:1]
        dinv = lax.rsqrt(deg)
        dinv_o[...] = dinv
        sl_o[...] = 1.0 / deg
        xa = x_r[...]
        ha = _mm(xa, w0a[...])
        hb = _mm(xa, w0b[...])
        h0_o[...] = ha
        h1_o[...] = hb
        hs0_o[...] = ha * dinv
        hs1_o[...] = hb * dinv

    dinv, sl, h_a, h_b, hs_a, hs_b = _rows_call(
        t0, [degp, xpd], [W0c[0], W0c[1]],
        [(_NPAD, 1), (_NPAD, 1), (_NPAD, 128), (_NPAD, 128),
         (_NPAD, 128), (_NPAD, 128)], _BR, _NPAD)

    agg0 = _sc_agg([hs_a, hs_b], srcw, dstw, z128)

    def _finish_mm2(aggp, ha, hb, dinv_a, sl_a, wra, wrb, bias,
                    n0_o, n1_o, y0_o, y1_o, ys0_o, ys1_o):
        dv = dinv_a[...]
        s = sl_a[...]
        b_ = bias[...]
        c0 = jax.nn.relu(dv * (aggp[0, 0] + aggp[1, 0]) + ha[...] * s
                         + b_[:, :128])
        c1 = jax.nn.relu(dv * (aggp[0, 1] + aggp[1, 1]) + hb[...] * s
                         + b_[:, 128:])
        n0_o[...] = c0
        n1_o[...] = c1
        y0 = _mm(c0, wra[...][:, :128]) + _mm(c1, wrb[...][:, :128])
        y1 = _mm(c0, wra[...][:, 128:]) + _mm(c1, wrb[...][:, 128:])
        y0_o[...] = y0
        y1_o[...] = y1
        ys0_o[...] = y0 * dv
        ys1_o[...] = y1 * dv

    h0_a, h0_b, y1_a, y1_b, y1s_a, y1s_b = _rows_call(
        _finish_mm2,
        [agg0, h_a, h_b, dinv, sl], [W1r[0], W1r[1], b0r],
        [(_NPAD, 128)] * 6, _BR, _NPAD)

    def _graph_call(fn, row_ins, full_ins, out_shapes):
        rs = []
        in_specs = []
        for a in row_ins:
            if a.ndim == 2:
                d = a.shape[1]
                rs.append(a.reshape(_G, _S, d))
                in_specs.append(pl.BlockSpec((1, _S, d),
                                             lambda i: (i, 0, 0)))
            else:
                two, C, n, D = a.shape
                rs.append(a.reshape(two, C, _G, _S, D))
                in_specs.append(pl.BlockSpec((two, C, 1, _S, D),
                                             lambda i: (0, 0, i, 0, 0)))
        in_specs += [_full_spec(a.shape) for a in full_ins]
        out_shape = []
        out_specs = []
        final = []
        for s in out_shapes:
            if s[0] == _G:
                out_shape.append(jax.ShapeDtypeStruct((_G, 1, s[1]),
                                                      jnp.float32))
                out_specs.append(pl.BlockSpec((1, 1, s[1]),
                                              lambda i: (i, 0, 0)))
                final.append((_G, s[1]))
            else:
                out_shape.append(jax.ShapeDtypeStruct((_G, _S, s[1]),
                                                      jnp.float32))
                out_specs.append(pl.BlockSpec((1, _S, s[1]),
                                              lambda i: (i, 0, 0)))
                final.append((s[0], s[1]))
        outs = pl.pallas_call(
            fn, grid=(_G,), in_specs=in_specs, out_specs=out_specs,
            out_shape=out_shape)(*rs, *full_ins)
        return tuple(o.reshape(f) for o, f in zip(outs, final))

    def tg(h0a, h0b, wfc, bfc_, w2b, cv_o):
        g0 = jnp.max(h0a[0], axis=0, keepdims=True)
        g1 = jnp.max(h0b[0], axis=0, keepdims=True)
        gi = (_mm(g0, wfc[...][:128]) + _mm(g1, wfc[...][128:]) + bfc_[...])
        cv_o[0] = _mm(gi, w2b[...])

    (cvec,) = _graph_call(tg, [h0_a[:_N], h0_b[:_N]],
                          [Wfc, bfc[None, :], W2b], [(_G, _HID)])
    cfull = jnp.zeros((_NPAD, _HID), f32).at[:_N].set(
        jnp.repeat(cvec, _S, axis=0))

    agg1 = _sc_agg([y1s_a, y1s_b], srcw, dstw, z128)
    h1_a, h1_b, y2_a, y2_b, y2s_a, y2s_b = _rows_call(
        _finish_mm2, [agg1, y1_a, y1_b, dinv, sl],
        [W1r[0], W1r[1], b1r], [(_NPAD, 128)] * 6, _BR, _NPAD)

    agg2 = _sc_agg([y2s_a, y2s_b], srcw, dstw, z128)

    def t3(aggp, ha, hb, dinv_a, sl_a, cf, wra, wrb, bias,
           y0_o, y1_o, ys0_o, ys1_o):
        dv = dinv_a[...]
        s = sl_a[...]
        b_ = bias[...]
        c0 = jax.nn.relu(dv * (aggp[0, 0] + aggp[1, 0]) + ha[...] * s
                         + b_[:, :128])
        c1 = jax.nn.relu(dv * (aggp[0, 1] + aggp[1, 1]) + hb[...] * s
                         + b_[:, 128:])
        cfl = cf[...]
        y0 = _mm(c0, wra[...][:, :128]) + _mm(c1, wrb[...][:, :128]) \
            + cfl[:, :128]
        y1 = _mm(c0, wra[...][:, 128:]) + _mm(c1, wrb[...][:, 128:]) \
            + cfl[:, 128:]
        y0_o[...] = y0
        y1_o[...] = y1
        ys0_o[...] = y0 * dv
        ys1_o[...] = y1 * dv

    h2p_a, h2p_b, h2ps_a, h2ps_b = _rows_call(
        t3, [agg2, y2_a, y2_b, dinv, sl, cfull],
        [W2ar[0], W2ar[1], b1r], [(_NPAD, 128)] * 4, _BR, _NPAD)

    agg3 = _sc_agg([h2ps_a, h2ps_b], srcw, dstw, z128)

    def t4b(aggp, ha, hb, dinv_a, sl_a, w3, bias, lg8_o, lg_o):
        dv = dinv_a[...]
        s = sl_a[...]
        b_ = bias[...]
        c0 = jax.nn.relu(dv * (aggp[0, 0] + aggp[1, 0]) + ha[...] * s
                         + b_[:, :128])
        c1 = jax.nn.relu(dv * (aggp[0, 1] + aggp[1, 1]) + hb[...] * s
                         + b_[:, 128:])
        lg = _mm(c0, w3[...][:128]) + _mm(c1, w3[...][128:])
        lg8_o[...] = jnp.broadcast_to(lg * dv, (lg.shape[0], 128))
        lg_o[...] = lg

    lg8, lg = _rows_call(t4b, [agg3, h2p_a, h2p_b, dinv, sl],
                         [W3, b2r], [(_NPAD, 128), (_NPAD, 1)], _BR, _NPAD)

    agg4 = _sc_agg([lg8], srcw, dstw, z128)

    def t5(aggp, lg_r, dinv_a, sl_a, gum_r, x_r, wq1a, wq1b, b3_,
           q1h0_o, q1h1_o, q1hs0_o, q1hs1_o):
        dv = dinv_a[0]
        ag = aggp[0, 0, 0] + aggp[1, 0, 0]
        logit = dv * ag[:, 0:1] + lg_r[0] * sl_a[0] + b3_[...]
        noisy = gum_r[0] + logit
        m = jnp.max(noisy, axis=0, keepdims=True)
        e = jnp.exp(noisy - m)
        se = jnp.sum(e, axis=0, keepdims=True)
        T = jnp.max(e / se, axis=1, keepdims=True)
        xm = x_r[0] * T
        qa = _mm(xm, wq1a[...])
        qb = _mm(xm, wq1b[...])
        q1h0_o[0] = qa
        q1h1_o[0] = qb
        q1hs0_o[0] = qa * dv
        q1hs1_o[0] = qb * dv

    q1h_a, q1h_b, q1hs_a, q1hs_b = _graph_call(
        t5, [agg4[:, :, :_N], lg[:_N], dinv[:_N], sl[:_N], gum, x],
        [Wq1c[0], Wq1c[1], b3[None, :]],
        [(_N, 128), (_N, 128), (_N, 128), (_N, 128)])
    q1hs_a = jnp.zeros((_NPAD, 128), f32).at[:_N].set(q1hs_a)
    q1hs_b = jnp.zeros((_NPAD, 128), f32).at[:_N].set(q1hs_b)

    agg5 = _sc_agg([q1hs_a, q1hs_b], srcw, dstw, z128)

    def t6(aggp, ha, hb, dinv_a, sl_a, bias, pp, wq2a, wq2b,
           keep8_o, hq2_o, keep_o):
        dv = dinv_a[0]
        s = sl_a[0]
        b_ = bias[...]
        c0 = dv * (aggp[0, 0, 0] + aggp[1, 0, 0]) + ha[0] * s + b_[:, :128]
        c1 = dv * (aggp[0, 1, 0] + aggp[1, 1, 0]) + hb[0] * s + b_[:, 128:]
        ppv = pp[...]
        pnorm = lax.rsqrt(jnp.sum(ppv * ppv))
        score = (_mm(c0, ppv[:128]) + _mm(c1, ppv[128:])) * pnorm

        bits = lax.bitcast_convert_type(score, jnp.uint32)
        key = jnp.where(bits >> 31 == jnp.uint32(1), ~bits,
                        bits | jnp.uint32(0x80000000))

        def bit_step(i, carry):
            prefix, need = carry
            b_pos = (31 - i).astype(jnp.uint32)
            cand = (prefix >> b_pos) | jnp.uint32(1)
            cnt = jnp.sum(jnp.where((key >> b_pos) == cand, 1, 0))
            take = cnt >= need
            prefix = jnp.where(take, prefix | (jnp.uint32(1) << b_pos),
                               prefix)
            need = jnp.where(take, need, need - cnt)
            return prefix, need

        prefix, _need = lax.fori_loop(
            0, 32, bit_step, (jnp.uint32(0), jnp.int32(_KP)))
        cnt_gt = jnp.sum(jnp.where(key > prefix, 1, 0))
        extra = _KP - cnt_gt
        idx = lax.broadcasted_iota(jnp.int32, key.shape, 0)
        tie = key == prefix

        def ib_step(i, carry):
            lo, hi = carry
            mid = (lo + hi) // 2
            c = jnp.sum(jnp.where(tie & (idx <= mid), 1, 0))
            good = c >= extra
            hi = jnp.where(good, mid, hi)
            lo = jnp.where(good, lo, mid + 1)
            return lo, hi

        lo, _hi = lax.fori_loop(0, 11, ib_step,
                                (jnp.int32(0), jnp.int32(_S - 1)))
        keep = jnp.where((key > prefix) | (tie & (idx <= lo)), 1.0, 0.0)
        keep_o[0] = keep[:, 0:1]
        keep8_o[0] = jnp.broadcast_to(keep[:, 0:1], (keep.shape[0], 128))
        w = jnp.tanh(score)
        xp0 = jax.nn.relu(c0 * w) * keep[:, 0:1]
        xp1 = jax.nn.relu(c1 * w) * keep[:, 0:1]
        hq2_o[0] = _mm(xp0, wq2a[...]) + _mm(xp1, wq2b[...])

    keep8, hq2, keep = _graph_call(
        t6, [agg5[:, :, :_N], q1h_a, q1h_b, dinv[:_N], sl[:_N]],
        [bq1p, ppad, Wq2r[0], Wq2r[1]],
        [(_N, 128), (_N, 32), (_N, 1)])
    keep8p = jnp.zeros((_NPAD, 128), f32).at[:_N].set(keep8)

    degq = _sc_agg([keep8p], srcw, dstw, z128)

    def t7(degq_r, hq2_r, keep_r, hq2s_o, dinv0_o, slp_o):
        kp = keep_r[...]
        sj = degq_r[0, 0, :, 0:1] + degq_r[1, 0, :, 0:1]
        dp = 1.0 + kp * sj
        dinv0 = kp * lax.rsqrt(dp)
        dinv0_o[...] = dinv0
        slp_o[...] = 1.0 / dp
        hq2s_o[...] = hq2_r[...] * dinv0

    hq2p = jnp.zeros((_NPAD, 128), f32).at[:_N, :32].set(hq2)
    keepp = jnp.zeros((_NPAD, 1), f32).at[:_N].set(keep)
    hq2sp, dinv0, slp = _rows_call(
        t7, [degq, hq2p, keepp], [],
        [(_NPAD, 128), (_NPAD, 1), (_NPAD, 1)], _BR, _NPAD)

    agg6 = _sc_agg([hq2sp], srcw, dstw, z128)

    def t8(aggp, hq2_r, dinv0_r, slp_r, keep_r, bq2_, wlin, blin_, out_o):
        dv0 = dinv0_r[0]
        q2 = jax.nn.relu(dv0 * (aggp[0, 0, 0] + aggp[1, 0, 0])[:, :32]
                         + hq2_r[0] * slp_r[0] + bq2_[...])
        q2 = q2 * keep_r[0]
        pooled = jnp.sum(q2, axis=0, keepdims=True) * (1.0 / _KP)
        out_o[0] = _mm(pooled, wlin[...]) + blin_[...]

    (out,) = _graph_call(
        t8, [agg6[:, :, :_N], hq2, dinv0[:_N], slp[:_N], keep],
        [bq2r, Wlin, blinr], [(_G, 10)])
    return out
